# baseline (speedup 1.0000x reference)
import functools

import jax
import jax.numpy as jnp
from jax import lax
from jax.experimental import pallas as pl
from jax.experimental.pallas import tpu as pltpu
from jax.experimental.pallas import tpu_sc as plsc

N = 10000
D = 128
E = 320000

NC = 2
NS = 16
NW = NC * NS

CHUNK = 128
CHUNKS_TOTAL = E // CHUNK
TILE_CHUNKS = CHUNKS_TOTAL // NW
EXTRA_BASE = NW * TILE_CHUNKS
ROWS_PER_TILE = 624
TAIL_ROW = NS * ROWS_PER_TILE
TAIL = N - TAIL_ROW
ZROWS = 8

BLOCK_M = 2000


def _fold_mm_body(x_ref, wpt_ref, w1t_ref, bp_ref, o_ref, wct_s, bc_s):
    @pl.when(pl.program_id(0) == 0)
    def _():
        wct_s[...] = jnp.dot(wpt_ref[...], w1t_ref[...],
                             preferred_element_type=jnp.float32)
        bc_s[...] = jnp.dot(bp_ref[...], w1t_ref[...],
                            preferred_element_type=jnp.float32)

    o_ref[...] = (
        jnp.dot(x_ref[...], wct_s[...], preferred_element_type=jnp.float32)
        + bc_s[...]
    )


def _fold_mm(x, wpt, w1t, bp):
    grid = (N // BLOCK_M,)
    return pl.pallas_call(
        _fold_mm_body,
        grid=grid,
        in_specs=[
            pl.BlockSpec((BLOCK_M, D), lambda i: (i, 0)),
            pl.BlockSpec((D, D), lambda i: (0, 0)),
            pl.BlockSpec((D, D), lambda i: (0, 0)),
            pl.BlockSpec((1, D), lambda i: (0, 0)),
        ],
        out_specs=pl.BlockSpec((BLOCK_M, D), lambda i: (i, 0)),
        out_shape=jax.ShapeDtypeStruct((N, D), jnp.float32),
        scratch_shapes=[
            pltpu.VMEM((D, D), jnp.float32),
            pltpu.VMEM((1, D), jnp.float32),
        ],
    )(x, wpt, w1t, bp.reshape(1, D))


def _agg_relu_mm_body(h_ref, p0_ref, p1_ref, w_ref, b_ref, o_ref):
    s = jnp.maximum(h_ref[...] + p0_ref[...] + p1_ref[...] + b_ref[...], 0.0)
    o_ref[...] = jnp.dot(s, w_ref[...], preferred_element_type=jnp.float32)


def _agg_relu_mm(h, parts, b1, w2t):
    grid = (N // BLOCK_M,)
    return pl.pallas_call(
        _agg_relu_mm_body,
        grid=grid,
        in_specs=[
            pl.BlockSpec((BLOCK_M, D), lambda i: (i, 0)),
            pl.BlockSpec((BLOCK_M, D), lambda i: (i, 0)),
            pl.BlockSpec((BLOCK_M, D), lambda i: (i, 0)),
            pl.BlockSpec((D, D), lambda i: (0, 0)),
            pl.BlockSpec((1, D), lambda i: (0, 0)),
        ],
        out_specs=pl.BlockSpec((BLOCK_M, D), lambda i: (i, 0)),
        out_shape=jax.ShapeDtypeStruct((N, D), jnp.float32),
    )(h, parts[0], parts[1], w2t, b1.reshape(1, D))


def _final_add_body(g_ref, q0_ref, q1_ref, b_ref, o_ref):
    o_ref[...] = g_ref[...] + q0_ref[...] + q1_ref[...] + b_ref[...]


def _final_add(g, parts, b2):
    grid = (N // BLOCK_M,)
    return pl.pallas_call(
        _final_add_body,
        grid=grid,
        in_specs=[
            pl.BlockSpec((BLOCK_M, D), lambda i: (i, 0)),
            pl.BlockSpec((BLOCK_M, D), lambda i: (i, 0)),
            pl.BlockSpec((BLOCK_M, D), lambda i: (i, 0)),
            pl.BlockSpec((1, D), lambda i: (0, 0)),
        ],
        out_specs=pl.BlockSpec((BLOCK_M, D), lambda i: (i, 0)),
        out_shape=jax.ShapeDtypeStruct((N, D), jnp.float32),
    )(g, parts[0], parts[1], b2.reshape(1, D))


def _sc_agg(h, edge_index):
    mesh = plsc.VectorSubcoreMesh(
        core_axis_name="c", subcore_axis_name="s", num_cores=NC, num_subcores=NS
    )

    @functools.partial(
        pl.kernel,
        out_type=(jax.ShapeDtypeStruct((N, D), jnp.float32),
                  jax.ShapeDtypeStruct((N, D), jnp.float32)),
        mesh=mesh,
        scratch_types=[
            [pltpu.VMEM((2, CHUNK), jnp.int32) for _ in range(3)],
            [pltpu.VMEM((CHUNK, D), jnp.float32) for _ in range(3)],
            pltpu.VMEM((ZROWS, D), jnp.float32),
            pltpu.VMEM_SHARED((N, D), jnp.float32),
            [pltpu.SemaphoreType.DMA for _ in range(3)],
            [pltpu.SemaphoreType.DMA for _ in range(3)],
            [pltpu.SemaphoreType.DMA for _ in range(3)],
        ],
    )
    def k(h_hbm, e_hbm, out0_hbm, out1_hbm, ib, rows, zeros_v, acc_sh,
          isem, gsem, ssem):
        cid = lax.axis_index("c")
        sid = lax.axis_index("s")

        @pl.loop(0, ZROWS)
        def _(r):
            @pl.loop(0, D, step=16)
            def _(j):
                zeros_v[r, pl.ds(j, 16)] = jnp.zeros((16,), jnp.float32)

        row0 = sid * ROWS_PER_TILE

        @pl.loop(0, ROWS_PER_TILE, step=ZROWS)
        def _(r0):
            pltpu.sync_copy(zeros_v, acc_sh.at[pl.ds(row0 + r0, ZROWS)])

        @pl.when(sid == NS - 1)
        def _():
            @pl.loop(0, TAIL, step=ZROWS)
            def _(r0):
                pltpu.sync_copy(zeros_v, acc_sh.at[pl.ds(TAIL_ROW + r0, ZROWS)])

        plsc.subcore_barrier()

        wid = cid * NS + sid
        base = wid * TILE_CHUNKS

        def _echunk(j):
            return e_hbm.at[pl.ds(0, 2), pl.ds((base + j) * CHUNK, CHUNK)]

        def _iload(j, m4):
            pltpu.async_copy(_echunk(j), ib[m4], isem[m4])

        def _iwait(j, m4):
            pltpu.make_async_copy(_echunk(j), ib[m4], isem[m4]).wait()

        def _gstart(j, m4, m2):
            pltpu.async_copy(h_hbm.at[ib[m4].at[0]], rows[m2], gsem[m2])

        def _gwait(j, m4, m2):
            pltpu.make_async_copy(h_hbm.at[ib[m4].at[0]], rows[m2],
                                  gsem[m2]).wait()

        def _sstart(j, m4, m2):
            pltpu.async_copy(rows[m2], acc_sh.at[ib[m4].at[1]], ssem[m2],
                             add=True)

        def _swait(j, m4, m2):
            pltpu.make_async_copy(rows[m2], acc_sh.at[ib[m4].at[1]],
                                  ssem[m2]).wait()

        def _step(j, jm, first=False, do_next=True, do_pref=True):
            m = jm % 3
            n = (jm + 1) % 3
            p = (jm + 2) % 3
            _gwait(j, m, m)
            _sstart(j, m, m)
            if not first:
                _swait(j - 1, p, p)
            if do_pref:
                _iload(j + 2, p)
            if do_next:
                _iwait(j + 1, n)
                _gstart(j + 1, n, n)

        _iload(0, 0)
        _iload(1, 1)
        _iwait(0, 0)
        _gstart(0, 0, 0)

        _step(0, 0, first=True)
        _step(1, 1)
        _step(2, 2)

        @pl.loop(0, (TILE_CHUNKS - 6) // 3)
        def _(g):
            j = 3 * g + 3
            _step(j + 0, 0)
            _step(j + 1, 1)
            _step(j + 2, 2)

        _step(TILE_CHUNKS - 3, (TILE_CHUNKS - 3) % 3)
        _step(TILE_CHUNKS - 2, (TILE_CHUNKS - 2) % 3, do_pref=False)
        _step(TILE_CHUNKS - 1, (TILE_CHUNKS - 1) % 3, do_next=False,
              do_pref=False)
        _swait(TILE_CHUNKS - 1, (TILE_CHUNKS - 1) % 3, (TILE_CHUNKS - 1) % 3)

        @pl.when(sid < 2)
        def _():
            ec = e_hbm.at[pl.ds(0, 2),
                          pl.ds((EXTRA_BASE + cid * 2 + sid) * CHUNK, CHUNK)]
            pltpu.sync_copy(ec, ib[0])
            pltpu.async_copy(h_hbm.at[ib[0].at[0]], rows[0], gsem[0]).wait()
            pltpu.sync_copy(rows[0], acc_sh.at[ib[0].at[1]], add=True)

        plsc.subcore_barrier()

        @pl.when(cid == 0)
        def _():
            pltpu.sync_copy(
                acc_sh.at[pl.ds(row0, ROWS_PER_TILE)],
                out0_hbm.at[pl.ds(row0, ROWS_PER_TILE)],
            )

            @pl.when(sid == NS - 1)
            def _():
                pltpu.sync_copy(
                    acc_sh.at[pl.ds(TAIL_ROW, TAIL)],
                    out0_hbm.at[pl.ds(TAIL_ROW, TAIL)],
                )

        @pl.when(cid == 1)
        def _():
            pltpu.sync_copy(
                acc_sh.at[pl.ds(row0, ROWS_PER_TILE)],
                out1_hbm.at[pl.ds(row0, ROWS_PER_TILE)],
            )

            @pl.when(sid == NS - 1)
            def _():
                pltpu.sync_copy(
                    acc_sh.at[pl.ds(TAIL_ROW, TAIL)],
                    out1_hbm.at[pl.ds(TAIL_ROW, TAIL)],
                )

    return k(h, edge_index)


@jax.jit
def kernel(x, edge_index, W_pre, b_pre, W1, b1, W2, b2):
    g1 = _fold_mm(x, W_pre.T, W1.T, b_pre)
    p = _sc_agg(g1, edge_index)
    g2 = _agg_relu_mm(g1, (p[0], p[1]), b1, W2.T)
    q = _sc_agg(g2, edge_index)
    return _final_add(g2, (q[0], q[1]), b2)
